# baseline (speedup 1.0000x reference)
import functools

import jax
import jax.numpy as jnp
from jax import lax
from jax.experimental import pallas as pl
from jax.experimental.pallas import tpu as pltpu
from jax.experimental.pallas import tpu_sc as plsc

DIM = 128
MAX_LEN = 2048
N_POS = 819200

_NC = 2
_NS = 16
_NW = _NC * _NS

_B_PER_W = N_POS // _NW
_G = 128
_NG = _B_PER_W // _G
_SC = 2
_NGS = _NG // _SC

_NBUF = 2
_LEAD = 1


def _make_sc_gather():
    mesh = plsc.VectorSubcoreMesh(core_axis_name="c", subcore_axis_name="s")

    @functools.partial(
        pl.kernel,
        mesh=mesh,
        out_type=jax.ShapeDtypeStruct((N_POS // _G, _G, DIM), jnp.float32),
        scratch_types=[
            pltpu.VMEM((_NG, _G), jnp.int32),
            pltpu.VMEM((_NBUF, _SC, _G, DIM), jnp.float32),
            pltpu.VMEM_SHARED((MAX_LEN, DIM), jnp.float32),
            [pltpu.SemaphoreType.DMA] * _NBUF,
            [pltpu.SemaphoreType.DMA] * _NBUF,
        ],
    )
    def body(table_hbm, pos_hbm, out_hbm, idx_v, rows_v, table_sp, gsems, ssems):
        wid = lax.axis_index("s") * _NC + lax.axis_index("c")
        base = wid * _NG

        sid = lax.axis_index("s")
        rpt = MAX_LEN // _NS
        pltpu.sync_copy(
            table_hbm.at[pl.ds(sid * rpt, rpt)],
            table_sp.at[pl.ds(sid * rpt, rpt)],
        )
        pltpu.sync_copy(pos_hbm.at[pl.ds(wid * _NG, _NG)], idx_v)
        plsc.subcore_barrier()

        def g_start(s, b):
            for h in range(_SC):
                pltpu.async_copy(
                    table_sp.at[idx_v.at[s * _SC + h]],
                    rows_v.at[b].at[h],
                    gsems[b],
                )

        def g_wait(s, b):
            pltpu.make_async_copy(
                out_hbm.at[pl.ds(0, _SC)], rows_v.at[b], gsems[b]
            ).wait()

        def s_copy(s, b):
            return pltpu.make_async_copy(
                rows_v.at[b], out_hbm.at[pl.ds(base + s * _SC, _SC)], ssems[b]
            )

        def visit(j, b, bl, do_swait, do_gstart, do_tail):
            if do_swait:
                s_copy(j - _NBUF, b).wait()
            if do_gstart:
                g_start(j, b)
            if do_tail:
                g_wait(j - _LEAD, bl)
                s_copy(j - _LEAD, bl).start()

        peel = _NBUF + (_NGS - _NBUF) % _NBUF
        for j in range(peel):
            visit(j, j % _NBUF, (j - _LEAD) % _NBUF,
                  j >= _NBUF, True, j >= _LEAD)

        def step(i, carry):
            j0 = peel + i * _NBUF
            for k in range(_NBUF):
                visit(j0 + k, (peel + k) % _NBUF, (peel + k - _LEAD) % _NBUF,
                      True, True, True)
            return carry

        lax.fori_loop(0, (_NGS - peel) // _NBUF, step, 0)

        for j in range(_NGS, _NGS + _LEAD):
            visit(j, j % _NBUF, (j - _LEAD) % _NBUF,
                  True, False, True)

        for c in range(_NGS + _LEAD - _NBUF, _NGS):
            s_copy(c, c % _NBUF).wait()

    return body


_sc_gather = _make_sc_gather()


def kernel(pe, pos):
    table = pe.reshape(MAX_LEN, DIM)
    pos2 = pos.reshape(N_POS // _G, _G)
    return _sc_gather(table, pos2).reshape(N_POS, DIM)


# speedup vs baseline: 1.0212x; 1.0212x over previous
import functools

import jax
import jax.numpy as jnp
from jax import lax
from jax.experimental import pallas as pl
from jax.experimental.pallas import tpu as pltpu
from jax.experimental.pallas import tpu_sc as plsc

DIM = 128
MAX_LEN = 2048
N_POS = 819200

_NC = 2
_NS = 16
_NW = _NC * _NS

_B_PER_W = N_POS // _NW
_G = 128
_NG = _B_PER_W // _G

_NBUF = 5
_LEAD = 2


def _make_sc_gather():
    mesh = plsc.VectorSubcoreMesh(core_axis_name="c", subcore_axis_name="s")

    @functools.partial(
        pl.kernel,
        mesh=mesh,
        out_type=jax.ShapeDtypeStruct((N_POS, DIM), jnp.float32),
        scratch_types=[
            pltpu.VMEM((_NG, _G), jnp.int32),
            pltpu.VMEM((_NBUF, _G, DIM), jnp.float32),
            pltpu.VMEM_SHARED((MAX_LEN, DIM), jnp.float32),
            [pltpu.SemaphoreType.DMA] * _NBUF,
            [pltpu.SemaphoreType.DMA] * _NBUF,
        ],
    )
    def body(table_hbm, pos_hbm, out_hbm, idx_v, rows_v, table_sp, gsems, ssems):
        wid = lax.axis_index("s") * _NC + lax.axis_index("c")
        base = wid * _B_PER_W

        sid = lax.axis_index("s")
        rows_per_tile = MAX_LEN // _NS
        pltpu.sync_copy(
            table_hbm.at[pl.ds(sid * rows_per_tile, rows_per_tile)],
            table_sp.at[pl.ds(sid * rows_per_tile, rows_per_tile)],
        )
        pltpu.sync_copy(pos_hbm.at[pl.ds(wid * _NG, _NG)], idx_v)
        plsc.subcore_barrier()

        def g_copy(j, b):
            return pltpu.make_async_copy(
                table_sp.at[idx_v.at[j]], rows_v.at[b], gsems[b]
            )

        def s_copy(j, b):
            return pltpu.make_async_copy(
                rows_v.at[b], out_hbm.at[pl.ds(base + j * _G, _G)], ssems[b]
            )

        for j in range(_NBUF):
            g_copy(j, j).start()
            if j >= _LEAD:
                c = j - _LEAD
                g_copy(c, c).wait()
                s_copy(c, c).start()

        def step(i, carry):
            j0 = _NBUF + i * _NBUF
            for k in range(_NBUF):
                j = j0 + k
                bl = (k - _LEAD) % _NBUF
                s_copy(j - _NBUF, k).wait()
                g_copy(j, k).start()
                g_copy(j - _LEAD, bl).wait()
                s_copy(j - _LEAD, bl).start()
            return carry

        lax.fori_loop(0, (_NG - _NBUF) // _NBUF, step, 0)

        for t in range(_LEAD):
            j = _NG + t
            bl = (j - _LEAD) % _NBUF
            s_copy(j - _NBUF, j % _NBUF).wait()
            g_copy(j - _LEAD, bl).wait()
            s_copy(j - _LEAD, bl).start()

        for c in range(_NG + _LEAD - _NBUF, _NG):
            s_copy(c, c % _NBUF).wait()

    return body


_sc_gather = _make_sc_gather()


def kernel(pe, pos):
    table = pe.reshape(MAX_LEN, DIM)
    pos2 = pos.reshape(N_POS // _G, _G)
    return _sc_gather(table, pos2)
